# Initial kernel scaffold; baseline (speedup 1.0000x reference)
import jax
import jax.numpy as jnp
from jax.experimental import pallas as pl


def kernel(x, edge_index, params):
    raise NotImplementedError("write your pallas kernel here")


# speedup vs baseline: 20.1765x; 20.1765x over previous
import functools

import jax
import jax.numpy as jnp
from jax import lax
from jax.experimental import pallas as pl
from jax.experimental.pallas import tpu as pltpu
from jax.experimental.pallas import tpu_sc as plsc

_N = 10000
_E = 320000
_D = 128
_EPS = 1e-5

_NC = 2
_NS = 16
_NW = _NC * _NS
_TRASH = 112
_NP = _N + _TRASH
_STRIPE = _NP // _NS
_K = 128
_RW = 80
_EROWS = _NW * _RW
_EPAD = _EROWS * _K

_mesh = plsc.VectorSubcoreMesh(
    core_axis_name="c", subcore_axis_name="s",
    num_cores=_NC, num_subcores=_NS)


@functools.partial(
    pl.kernel,
    out_type=jax.ShapeDtypeStruct((_NC * _NP, _D), jnp.float32),
    mesh=_mesh,
    scratch_types=[
        pltpu.VMEM((_RW, _K), jnp.int32),
        pltpu.VMEM((_K, _D), jnp.float32),
        pltpu.VMEM_SHARED((_NP, _D), jnp.float32),
    ],
)
def _deg_kernel(dst_hbm, ones_hbm, z_hbm, out_hbm, dst_v, ones_v, acc):
    c = lax.axis_index("c")
    s = lax.axis_index("s")
    pltpu.sync_copy(z_hbm.at[pl.ds(s * _STRIPE, _STRIPE)],
                    acc.at[pl.ds(s * _STRIPE, _STRIPE)])
    pltpu.sync_copy(ones_hbm, ones_v)
    w = c * _NS + s
    pltpu.sync_copy(dst_hbm.at[pl.ds(w * _RW, _RW)], dst_v)
    plsc.subcore_barrier()

    def body(g, carry):
        pltpu.sync_copy(ones_v, acc.at[dst_v.at[g]], add=True)
        return carry

    lax.fori_loop(0, _RW, body, 0)
    plsc.subcore_barrier()
    pltpu.sync_copy(acc.at[pl.ds(s * _STRIPE, _STRIPE)],
                    out_hbm.at[pl.ds(c * _NP + s * _STRIPE, _STRIPE)])


@functools.partial(
    pl.kernel,
    out_type=jax.ShapeDtypeStruct((_NC * _NP, _D), jnp.float32),
    mesh=_mesh,
    scratch_types=[
        pltpu.VMEM((_RW, _K), jnp.int32),
        pltpu.VMEM((_RW, _K), jnp.int32),
        pltpu.VMEM((_K, _D), jnp.float32),
        pltpu.VMEM_SHARED((_NP, _D), jnp.float32),
        pltpu.SemaphoreType.DMA,
    ],
)
def _scatter_kernel(m_hbm, src_hbm, dst_hbm, z_hbm, out_hbm,
                    src_v, dst_v, rows_v, acc, sem):
    c = lax.axis_index("c")
    s = lax.axis_index("s")
    pltpu.sync_copy(z_hbm.at[pl.ds(s * _STRIPE, _STRIPE)],
                    acc.at[pl.ds(s * _STRIPE, _STRIPE)])
    w = c * _NS + s
    pltpu.sync_copy(src_hbm.at[pl.ds(w * _RW, _RW)], src_v)
    pltpu.sync_copy(dst_hbm.at[pl.ds(w * _RW, _RW)], dst_v)
    plsc.subcore_barrier()

    def body(g, carry):
        pltpu.async_copy(m_hbm.at[src_v.at[g]], rows_v, sem).wait()
        pltpu.sync_copy(rows_v, acc.at[dst_v.at[g]], add=True)
        return carry

    lax.fori_loop(0, _RW, body, 0)
    plsc.subcore_barrier()
    pltpu.sync_copy(acc.at[pl.ds(s * _STRIPE, _STRIPE)],
                    out_hbm.at[pl.ds(c * _NP + s * _STRIPE, _STRIPE)])


def _tc1_body(x_ref, fcw_ref, fcb_ref, c1w_ref, bn0g_ref, bn0b_ref,
              degp_ref, h0_ref, m1_ref, dinv_ref):
    x = x_ref[...]
    h = jnp.dot(x, fcw_ref[...], preferred_element_type=jnp.float32)
    h = h + fcb_ref[...]
    mu = jnp.mean(h, axis=0, keepdims=True)
    xc = h - mu
    var = jnp.mean(xc * xc, axis=0, keepdims=True)
    h0 = jax.nn.relu(bn0g_ref[...] * xc * lax.rsqrt(var + _EPS)
                     + bn0b_ref[...])
    degp = degp_ref[...]
    deg = degp[0:_N, 0:1] + degp[_NP:_NP + _N, 0:1] + 1.0
    dinv = lax.rsqrt(deg)
    m1 = jnp.dot(dinv * h0, c1w_ref[...], preferred_element_type=jnp.float32)
    h0_ref[...] = h0
    m1_ref[...] = m1
    dinv_ref[...] = dinv


_tc1 = pl.pallas_call(
    _tc1_body,
    out_shape=(
        jax.ShapeDtypeStruct((_N, _D), jnp.float32),
        jax.ShapeDtypeStruct((_N, _D), jnp.float32),
        jax.ShapeDtypeStruct((_N, 1), jnp.float32),
    ),
)


def _tc2_body(accp_ref, m1_ref, dinv_ref, h0_ref, c1b_ref, bn1g_ref,
              bn1b_ref, c2w_ref, m2_ref):
    a = accp_ref[...]
    agg = a[0:_N] + a[_NP:_NP + _N]
    dinv = dinv_ref[...]
    t = dinv * (agg + m1_ref[...]) + c1b_ref[...]
    mu = jnp.mean(t, axis=0, keepdims=True)
    xc = t - mu
    var = jnp.mean(xc * xc, axis=0, keepdims=True)
    h1 = jax.nn.relu(bn1g_ref[...] * xc * lax.rsqrt(var + _EPS)
                     + bn1b_ref[...]) + h0_ref[...]
    m2_ref[...] = jnp.dot(dinv * h1, c2w_ref[...],
                          preferred_element_type=jnp.float32)


_tc2 = pl.pallas_call(
    _tc2_body,
    out_shape=jax.ShapeDtypeStruct((_N, _D), jnp.float32),
)


def _tc3_body(accp_ref, m2_ref, dinv_ref, h0_ref, c2b_ref, bn2g_ref,
              bn2b_ref, out_ref):
    a = accp_ref[...]
    agg = a[0:_N] + a[_NP:_NP + _N]
    dinv = dinv_ref[...]
    t = dinv * (agg + m2_ref[...]) + c2b_ref[...]
    mu = jnp.mean(t, axis=0, keepdims=True)
    xc = t - mu
    var = jnp.mean(xc * xc, axis=0, keepdims=True)
    out_ref[...] = jax.nn.relu(bn2g_ref[...] * xc * lax.rsqrt(var + _EPS)
                               + bn2b_ref[...]) + h0_ref[...]


_tc3 = pl.pallas_call(
    _tc3_body,
    out_shape=jax.ShapeDtypeStruct((_N, _D), jnp.float32),
)


def kernel(x, edge_index, params):
    src = edge_index[0]
    dst = edge_index[1]
    pad = _EPAD - _E
    padi = jnp.arange(pad, dtype=jnp.int32)
    src2d = jnp.concatenate([src, padi % _TRASH]).reshape(_EROWS, _K)
    dst2d = jnp.concatenate(
        [dst, _N + (padi % _TRASH)]).reshape(_EROWS, _K)
    ones_m = jnp.ones((_K, _D), jnp.float32)
    zmat = jnp.zeros((_NP, _D), jnp.float32)

    p = params
    row = lambda v: v.reshape(1, _D)

    degp = _deg_kernel(dst2d, ones_m, zmat)
    h0, m1, dinv = _tc1(x, p['fc_w'], row(p['fc_b']), p['conv1_w'],
                        row(p['bn0_g']), row(p['bn0_b']), degp)
    acc1 = _scatter_kernel(m1, src2d, dst2d, zmat)
    m2 = _tc2(acc1, m1, dinv, h0, row(p['conv1_b']), row(p['bn1_g']),
              row(p['bn1_b']), p['conv2_w'])
    acc2 = _scatter_kernel(m2, src2d, dst2d, zmat)
    out = _tc3(acc2, m2, dinv, h0, row(p['conv2_b']), row(p['bn2_g']),
               row(p['bn2_b']))
    return out


# speedup vs baseline: 21.5692x; 1.0690x over previous
import functools

import jax
import jax.numpy as jnp
from jax import lax
from jax.experimental import pallas as pl
from jax.experimental.pallas import tpu as pltpu
from jax.experimental.pallas import tpu_sc as plsc

_N = 10000
_E = 320000
_D = 128
_EPS = 1e-5

_NC = 2
_NS = 16
_NW = _NC * _NS
_TRASH = 112
_NP = _N + _TRASH
_STRIPE = _NP // _NS
_K = 128
_RW = 80
_RH = 40
_EROWS = _NW * _RW
_EPAD = _EROWS * _K

_mesh = plsc.VectorSubcoreMesh(
    core_axis_name="c", subcore_axis_name="s",
    num_cores=_NC, num_subcores=_NS)


@functools.partial(
    pl.kernel,
    out_type=jax.ShapeDtypeStruct((_NC * _NP, _D), jnp.float32),
    mesh=_mesh,
    scratch_types=[
        pltpu.VMEM((_RH, _K), jnp.int32),
        pltpu.VMEM((_RH, _K), jnp.int32),
        pltpu.VMEM((_K, _D), jnp.float32),
        pltpu.VMEM((_K, _D), jnp.float32),
        pltpu.VMEM_SHARED((_NP, _D), jnp.float32),
        pltpu.SemaphoreType.DMA,
        pltpu.SemaphoreType.DMA,
    ],
)
def _scatter_kernel(m_hbm, src_hbm, dst_hbm, z_hbm, out_hbm,
                    src_v, dst_v, rows0, rows1, acc, sem0, sem1):
    c = lax.axis_index("c")
    s = lax.axis_index("s")
    pltpu.sync_copy(z_hbm.at[pl.ds(s * _STRIPE, _STRIPE)],
                    acc.at[pl.ds(s * _STRIPE, _STRIPE)])
    w = c * _NS + s
    plsc.subcore_barrier()

    for h in range(_RW // _RH):
        base = w * _RW + h * _RH
        pltpu.sync_copy(src_hbm.at[pl.ds(base, _RH)], src_v)
        pltpu.sync_copy(dst_hbm.at[pl.ds(base, _RH)], dst_v)
        pltpu.async_copy(m_hbm.at[src_v.at[0]], rows0, sem0)

        def pair(i, carry):
            g = 2 * i
            pltpu.make_async_copy(m_hbm.at[src_v.at[g]], rows0, sem0).wait()
            d1 = pltpu.async_copy(m_hbm.at[src_v.at[g + 1]], rows1, sem1)
            pltpu.sync_copy(rows0, acc.at[dst_v.at[g]], add=True)
            d1.wait()

            @pl.when(i < _RH // 2 - 1)
            def _():
                pltpu.async_copy(m_hbm.at[src_v.at[g + 2]], rows0, sem0)

            pltpu.sync_copy(rows1, acc.at[dst_v.at[g + 1]], add=True)
            return carry

        lax.fori_loop(0, _RH // 2, pair, 0)
    plsc.subcore_barrier()
    pltpu.sync_copy(acc.at[pl.ds(s * _STRIPE, _STRIPE)],
                    out_hbm.at[pl.ds(c * _NP + s * _STRIPE, _STRIPE)])


def _tc1_body(x_ref, fcw_ref, fcb_ref, c1w_ref, bn0g_ref, bn0b_ref,
              degp_ref, h0_ref, m1_ref, dinv_ref):
    x = x_ref[...]
    h = jnp.dot(x, fcw_ref[...], preferred_element_type=jnp.float32)
    h = h + fcb_ref[...]
    mu = jnp.mean(h, axis=0, keepdims=True)
    xc = h - mu
    var = jnp.mean(xc * xc, axis=0, keepdims=True)
    h0 = jax.nn.relu(bn0g_ref[...] * xc * lax.rsqrt(var + _EPS)
                     + bn0b_ref[...])
    degp = degp_ref[...]
    deg = degp[0:_N, 0:1] + degp[_NP:_NP + _N, 0:1] + 1.0
    dinv = lax.rsqrt(deg)
    m1 = jnp.dot(dinv * h0, c1w_ref[...], preferred_element_type=jnp.float32)
    h0_ref[...] = h0
    m1_ref[...] = m1
    dinv_ref[...] = dinv


_tc1 = pl.pallas_call(
    _tc1_body,
    out_shape=(
        jax.ShapeDtypeStruct((_N, _D), jnp.float32),
        jax.ShapeDtypeStruct((_N, _D), jnp.float32),
        jax.ShapeDtypeStruct((_N, 1), jnp.float32),
    ),
)


def _tc2_body(accp_ref, m1_ref, dinv_ref, h0_ref, c1b_ref, bn1g_ref,
              bn1b_ref, c2w_ref, m2_ref):
    a = accp_ref[...]
    agg = a[0:_N] + a[_NP:_NP + _N]
    dinv = dinv_ref[...]
    t = dinv * (agg + m1_ref[...]) + c1b_ref[...]
    mu = jnp.mean(t, axis=0, keepdims=True)
    xc = t - mu
    var = jnp.mean(xc * xc, axis=0, keepdims=True)
    h1 = jax.nn.relu(bn1g_ref[...] * xc * lax.rsqrt(var + _EPS)
                     + bn1b_ref[...]) + h0_ref[...]
    m2_ref[...] = jnp.dot(dinv * h1, c2w_ref[...],
                          preferred_element_type=jnp.float32)


_tc2 = pl.pallas_call(
    _tc2_body,
    out_shape=jax.ShapeDtypeStruct((_N, _D), jnp.float32),
)


def _tc3_body(accp_ref, m2_ref, dinv_ref, h0_ref, c2b_ref, bn2g_ref,
              bn2b_ref, out_ref):
    a = accp_ref[...]
    agg = a[0:_N] + a[_NP:_NP + _N]
    dinv = dinv_ref[...]
    t = dinv * (agg + m2_ref[...]) + c2b_ref[...]
    mu = jnp.mean(t, axis=0, keepdims=True)
    xc = t - mu
    var = jnp.mean(xc * xc, axis=0, keepdims=True)
    out_ref[...] = jax.nn.relu(bn2g_ref[...] * xc * lax.rsqrt(var + _EPS)
                               + bn2b_ref[...]) + h0_ref[...]


_tc3 = pl.pallas_call(
    _tc3_body,
    out_shape=jax.ShapeDtypeStruct((_N, _D), jnp.float32),
)


def kernel(x, edge_index, params):
    src = edge_index[0]
    dst = edge_index[1]
    pad = _EPAD - _E
    padi = jnp.arange(pad, dtype=jnp.int32)
    src2d = jnp.concatenate([src, padi % _TRASH]).reshape(_EROWS, _K)
    dst2d = jnp.concatenate(
        [dst, _N + (padi % _TRASH)]).reshape(_EROWS, _K)
    ones_nd = jnp.ones((_N, _D), jnp.float32)
    zmat = jnp.zeros((_NP, _D), jnp.float32)
    srcspread2d = (jnp.arange(_EPAD, dtype=jnp.int32) % _N).reshape(
        _EROWS, _K)

    p = params
    row = lambda v: v.reshape(1, _D)

    degp = _scatter_kernel(ones_nd, srcspread2d, dst2d, zmat)
    h0, m1, dinv = _tc1(x, p['fc_w'], row(p['fc_b']), p['conv1_w'],
                        row(p['bn0_g']), row(p['bn0_b']), degp)
    acc1 = _scatter_kernel(m1, src2d, dst2d, zmat)
    m2 = _tc2(acc1, m1, dinv, h0, row(p['conv1_b']), row(p['bn1_g']),
              row(p['bn1_b']), p['conv2_w'])
    acc2 = _scatter_kernel(m2, src2d, dst2d, zmat)
    out = _tc3(acc2, m2, dinv, h0, row(p['conv2_b']), row(p['bn2_g']),
               row(p['bn2_b']))
    return out


# speedup vs baseline: 24.5083x; 1.1363x over previous
import functools

import jax
import jax.numpy as jnp
from jax import lax
from jax.experimental import pallas as pl
from jax.experimental.pallas import tpu as pltpu
from jax.experimental.pallas import tpu_sc as plsc

_N = 10000
_E = 320000
_D = 128
_EPS = 1e-5

_NC = 2
_NS = 16
_NW = _NC * _NS
_TRASH = 112
_NP = _N + _TRASH
_STRIPE = _NP // _NS
_K = 128
_RW = 80
_RH = 40
_EROWS = _NW * _RW
_EPAD = _EROWS * _K

_mesh = plsc.VectorSubcoreMesh(
    core_axis_name="c", subcore_axis_name="s",
    num_cores=_NC, num_subcores=_NS)


@functools.partial(
    pl.kernel,
    out_type=jax.ShapeDtypeStruct((_NC * _NP, _D), jnp.float32),
    mesh=_mesh,
    scratch_types=[
        pltpu.VMEM((_RW, _K), jnp.int32),
        pltpu.VMEM((_K, _D), jnp.float32),
        pltpu.VMEM_SHARED((_NP, _D), jnp.float32),
        pltpu.SemaphoreType.DMA,
        pltpu.SemaphoreType.DMA,
        pltpu.SemaphoreType.DMA,
    ],
)
def _deg_kernel(dst_hbm, ones_hbm, z_hbm, out_hbm, dst_v, ones_v, acc,
                dsem0, dsem1, dsem2):
    c = lax.axis_index("c")
    s = lax.axis_index("s")
    pltpu.sync_copy(z_hbm.at[pl.ds(s * _STRIPE, _STRIPE)],
                    acc.at[pl.ds(s * _STRIPE, _STRIPE)])
    pltpu.sync_copy(ones_hbm, ones_v)
    w = c * _NS + s
    pltpu.sync_copy(dst_hbm.at[pl.ds(w * _RW, _RW)], dst_v)
    plsc.subcore_barrier()

    sems = (dsem0, dsem1, dsem2)
    nround = _RW // 3

    def rnd(i, carry):
        for k in range(3):
            g = i * 3 + k

            @pl.when(i > 0)
            def _():
                pltpu.make_async_copy(ones_v, acc.at[dst_v.at[g]],
                                      sems[k]).wait()

            pltpu.async_copy(ones_v, acc.at[dst_v.at[g]], sems[k], add=True)
        return carry

    lax.fori_loop(0, nround, rnd, 0)
    for k in range(3):
        g = (nround - 1) * 3 + k
        pltpu.make_async_copy(ones_v, acc.at[dst_v.at[g]], sems[k]).wait()
    for g in range(nround * 3, _RW):
        pltpu.sync_copy(ones_v, acc.at[dst_v.at[g]], add=True)
    plsc.subcore_barrier()
    pltpu.sync_copy(acc.at[pl.ds(s * _STRIPE, _STRIPE)],
                    out_hbm.at[pl.ds(c * _NP + s * _STRIPE, _STRIPE)])


@functools.partial(
    pl.kernel,
    out_type=jax.ShapeDtypeStruct((_NC * _NP, _D), jnp.float32),
    mesh=_mesh,
    scratch_types=[
        pltpu.VMEM((_RH, _K), jnp.int32),
        pltpu.VMEM((_RH, _K), jnp.int32),
        pltpu.VMEM((_K, _D), jnp.float32),
        pltpu.VMEM((_K, _D), jnp.float32),
        pltpu.VMEM_SHARED((_NP, _D), jnp.float32),
        pltpu.SemaphoreType.DMA,
        pltpu.SemaphoreType.DMA,
    ],
)
def _scatter_kernel(m_hbm, src_hbm, dst_hbm, z_hbm, out_hbm,
                    src_v, dst_v, rows0, rows1, acc, sem0, sem1):
    c = lax.axis_index("c")
    s = lax.axis_index("s")
    pltpu.sync_copy(z_hbm.at[pl.ds(s * _STRIPE, _STRIPE)],
                    acc.at[pl.ds(s * _STRIPE, _STRIPE)])
    w = c * _NS + s
    plsc.subcore_barrier()

    for h in range(_RW // _RH):
        base = w * _RW + h * _RH
        pltpu.sync_copy(src_hbm.at[pl.ds(base, _RH)], src_v)
        pltpu.sync_copy(dst_hbm.at[pl.ds(base, _RH)], dst_v)
        pltpu.async_copy(m_hbm.at[src_v.at[0]], rows0, sem0)

        def pair(i, carry):
            g = 2 * i
            pltpu.make_async_copy(m_hbm.at[src_v.at[g]], rows0, sem0).wait()
            d1 = pltpu.async_copy(m_hbm.at[src_v.at[g + 1]], rows1, sem1)
            pltpu.sync_copy(rows0, acc.at[dst_v.at[g]], add=True)
            d1.wait()

            @pl.when(i < _RH // 2 - 1)
            def _():
                pltpu.async_copy(m_hbm.at[src_v.at[g + 2]], rows0, sem0)

            pltpu.sync_copy(rows1, acc.at[dst_v.at[g + 1]], add=True)
            return carry

        lax.fori_loop(0, _RH // 2, pair, 0)
    plsc.subcore_barrier()
    pltpu.sync_copy(acc.at[pl.ds(s * _STRIPE, _STRIPE)],
                    out_hbm.at[pl.ds(c * _NP + s * _STRIPE, _STRIPE)])


def _tc1_body(x_ref, fcw_ref, fcb_ref, c1w_ref, bn0g_ref, bn0b_ref,
              degp_ref, h0_ref, m1_ref, dinv_ref):
    x = x_ref[...]
    h = jnp.dot(x, fcw_ref[...], preferred_element_type=jnp.float32)
    h = h + fcb_ref[...]
    mu = jnp.mean(h, axis=0, keepdims=True)
    xc = h - mu
    var = jnp.mean(xc * xc, axis=0, keepdims=True)
    h0 = jax.nn.relu(bn0g_ref[...] * xc * lax.rsqrt(var + _EPS)
                     + bn0b_ref[...])
    degp = degp_ref[...]
    deg = degp[0:_N, 0:1] + degp[_NP:_NP + _N, 0:1] + 1.0
    dinv = lax.rsqrt(deg)
    m1 = jnp.dot(dinv * h0, c1w_ref[...], preferred_element_type=jnp.float32)
    h0_ref[...] = h0
    m1_ref[...] = m1
    dinv_ref[...] = dinv


_tc1 = pl.pallas_call(
    _tc1_body,
    out_shape=(
        jax.ShapeDtypeStruct((_N, _D), jnp.float32),
        jax.ShapeDtypeStruct((_N, _D), jnp.float32),
        jax.ShapeDtypeStruct((_N, 1), jnp.float32),
    ),
)


def _tc2_body(accp_ref, m1_ref, dinv_ref, h0_ref, c1b_ref, bn1g_ref,
              bn1b_ref, c2w_ref, m2_ref):
    a = accp_ref[...]
    agg = a[0:_N] + a[_NP:_NP + _N]
    dinv = dinv_ref[...]
    t = dinv * (agg + m1_ref[...]) + c1b_ref[...]
    mu = jnp.mean(t, axis=0, keepdims=True)
    xc = t - mu
    var = jnp.mean(xc * xc, axis=0, keepdims=True)
    h1 = jax.nn.relu(bn1g_ref[...] * xc * lax.rsqrt(var + _EPS)
                     + bn1b_ref[...]) + h0_ref[...]
    m2_ref[...] = jnp.dot(dinv * h1, c2w_ref[...],
                          preferred_element_type=jnp.float32)


_tc2 = pl.pallas_call(
    _tc2_body,
    out_shape=jax.ShapeDtypeStruct((_N, _D), jnp.float32),
)


def _tc3_body(accp_ref, m2_ref, dinv_ref, h0_ref, c2b_ref, bn2g_ref,
              bn2b_ref, out_ref):
    a = accp_ref[...]
    agg = a[0:_N] + a[_NP:_NP + _N]
    dinv = dinv_ref[...]
    t = dinv * (agg + m2_ref[...]) + c2b_ref[...]
    mu = jnp.mean(t, axis=0, keepdims=True)
    xc = t - mu
    var = jnp.mean(xc * xc, axis=0, keepdims=True)
    out_ref[...] = jax.nn.relu(bn2g_ref[...] * xc * lax.rsqrt(var + _EPS)
                               + bn2b_ref[...]) + h0_ref[...]


_tc3 = pl.pallas_call(
    _tc3_body,
    out_shape=jax.ShapeDtypeStruct((_N, _D), jnp.float32),
)


def kernel(x, edge_index, params):
    src = edge_index[0]
    dst = edge_index[1]
    pad = _EPAD - _E
    padi = jnp.arange(pad, dtype=jnp.int32)
    src2d = jnp.concatenate([src, padi % _TRASH]).reshape(_EROWS, _K)
    dst2d = jnp.concatenate(
        [dst, _N + (padi % _TRASH)]).reshape(_EROWS, _K)
    ones_m = jnp.ones((_K, _D), jnp.float32)
    zmat = jnp.zeros((_NP, _D), jnp.float32)

    p = params
    row = lambda v: v.reshape(1, _D)

    degp = _deg_kernel(dst2d, ones_m, zmat)
    h0, m1, dinv = _tc1(x, p['fc_w'], row(p['fc_b']), p['conv1_w'],
                        row(p['bn0_g']), row(p['bn0_b']), degp)
    acc1 = _scatter_kernel(m1, src2d, dst2d, zmat)
    m2 = _tc2(acc1, m1, dinv, h0, row(p['conv1_b']), row(p['bn1_g']),
              row(p['bn1_b']), p['conv2_w'])
    acc2 = _scatter_kernel(m2, src2d, dst2d, zmat)
    out = _tc3(acc2, m2, dinv, h0, row(p['conv2_b']), row(p['bn2_g']),
               row(p['bn2_b']))
    return out


# speedup vs baseline: 24.6717x; 1.0067x over previous
import functools

import jax
import jax.numpy as jnp
from jax import lax
from jax.experimental import pallas as pl
from jax.experimental.pallas import tpu as pltpu
from jax.experimental.pallas import tpu_sc as plsc

_N = 10000
_E = 320000
_D = 128
_EPS = 1e-5

_NC = 2
_NS = 16
_NW = _NC * _NS
_TRASH = 112
_NP = _N + _TRASH
_STRIPE = _NP // _NS
_K = 128
_RW = 80
_RH = 40
_EROWS = _NW * _RW
_EPAD = _EROWS * _K

_mesh = plsc.VectorSubcoreMesh(
    core_axis_name="c", subcore_axis_name="s",
    num_cores=_NC, num_subcores=_NS)


@functools.partial(
    pl.kernel,
    out_type=jax.ShapeDtypeStruct((_NC * _NP, _D), jnp.float32),
    mesh=_mesh,
    scratch_types=[
        pltpu.VMEM((_RW, _K), jnp.int32),
        pltpu.VMEM((_K, _D), jnp.float32),
        pltpu.VMEM_SHARED((_NP, _D), jnp.float32),
        pltpu.SemaphoreType.DMA,
        pltpu.SemaphoreType.DMA,
        pltpu.SemaphoreType.DMA,
    ],
)
def _deg_kernel(dst_hbm, ones_hbm, z_hbm, out_hbm, dst_v, ones_v, acc,
                dsem0, dsem1, dsem2):
    c = lax.axis_index("c")
    s = lax.axis_index("s")
    pltpu.sync_copy(z_hbm.at[pl.ds(s * _STRIPE, _STRIPE)],
                    acc.at[pl.ds(s * _STRIPE, _STRIPE)])
    pltpu.sync_copy(ones_hbm, ones_v)
    w = c * _NS + s
    pltpu.sync_copy(dst_hbm.at[pl.ds(w * _RW, _RW)], dst_v)
    plsc.subcore_barrier()

    sems = (dsem0, dsem1, dsem2)
    nround = _RW // 3

    def rnd(i, carry):
        for k in range(3):
            g = i * 3 + k

            @pl.when(i > 0)
            def _():
                pltpu.make_async_copy(ones_v, acc.at[dst_v.at[g]],
                                      sems[k]).wait()

            pltpu.async_copy(ones_v, acc.at[dst_v.at[g]], sems[k], add=True)
        return carry

    lax.fori_loop(0, nround, rnd, 0)
    for k in range(3):
        g = (nround - 1) * 3 + k
        pltpu.make_async_copy(ones_v, acc.at[dst_v.at[g]], sems[k]).wait()
    for g in range(nround * 3, _RW):
        pltpu.sync_copy(ones_v, acc.at[dst_v.at[g]], add=True)
    plsc.subcore_barrier()
    pltpu.sync_copy(acc.at[pl.ds(s * _STRIPE, _STRIPE)],
                    out_hbm.at[pl.ds(c * _NP + s * _STRIPE, _STRIPE)])


@functools.partial(
    pl.kernel,
    out_type=jax.ShapeDtypeStruct((_NC * _NP, _D), jnp.float32),
    mesh=_mesh,
    scratch_types=[
        pltpu.VMEM((_RH, _K), jnp.int32),
        pltpu.VMEM((_RH, _K), jnp.int32),
        pltpu.VMEM((_K, _D), jnp.float32),
        pltpu.VMEM((_K, _D), jnp.float32),
        pltpu.VMEM_SHARED((_NP, _D), jnp.float32),
        pltpu.SemaphoreType.DMA,
        pltpu.SemaphoreType.DMA,
        pltpu.SemaphoreType.DMA,
        pltpu.SemaphoreType.DMA,
    ],
)
def _scatter_kernel(m_hbm, src_hbm, dst_hbm, z_hbm, out_hbm,
                    src_v, dst_v, rows0, rows1, acc,
                    gsem0, gsem1, ssem0, ssem1):
    c = lax.axis_index("c")
    s = lax.axis_index("s")
    pltpu.sync_copy(z_hbm.at[pl.ds(s * _STRIPE, _STRIPE)],
                    acc.at[pl.ds(s * _STRIPE, _STRIPE)])
    w = c * _NS + s
    plsc.subcore_barrier()

    for h in range(_RW // _RH):
        base = w * _RW + h * _RH
        pltpu.sync_copy(src_hbm.at[pl.ds(base, _RH)], src_v)
        pltpu.sync_copy(dst_hbm.at[pl.ds(base, _RH)], dst_v)
        pltpu.async_copy(m_hbm.at[src_v.at[0]], rows0, gsem0)

        def pair(i, carry):
            g = 2 * i
            pltpu.make_async_copy(m_hbm.at[src_v.at[g]], rows0, gsem0).wait()
            pltpu.async_copy(rows0, acc.at[dst_v.at[g]], ssem0, add=True)

            @pl.when(i > 0)
            def _():
                pltpu.make_async_copy(rows1, acc.at[dst_v.at[g - 1]],
                                      ssem1).wait()

            pltpu.async_copy(m_hbm.at[src_v.at[g + 1]], rows1, gsem1)
            pltpu.make_async_copy(m_hbm.at[src_v.at[g + 1]], rows1,
                                  gsem1).wait()
            pltpu.async_copy(rows1, acc.at[dst_v.at[g + 1]], ssem1, add=True)
            pltpu.make_async_copy(rows0, acc.at[dst_v.at[g]], ssem0).wait()

            @pl.when(i < _RH // 2 - 1)
            def _():
                pltpu.async_copy(m_hbm.at[src_v.at[g + 2]], rows0, gsem0)

            return carry

        lax.fori_loop(0, _RH // 2, pair, 0)
        pltpu.make_async_copy(rows1, acc.at[dst_v.at[_RH - 1]], ssem1).wait()
    plsc.subcore_barrier()
    pltpu.sync_copy(acc.at[pl.ds(s * _STRIPE, _STRIPE)],
                    out_hbm.at[pl.ds(c * _NP + s * _STRIPE, _STRIPE)])


def _tc1_body(x_ref, fcw_ref, fcb_ref, c1w_ref, bn0g_ref, bn0b_ref,
              degp_ref, h0_ref, m1_ref, dinv_ref):
    x = x_ref[...]
    h = jnp.dot(x, fcw_ref[...], preferred_element_type=jnp.float32)
    h = h + fcb_ref[...]
    mu = jnp.mean(h, axis=0, keepdims=True)
    xc = h - mu
    var = jnp.mean(xc * xc, axis=0, keepdims=True)
    h0 = jax.nn.relu(bn0g_ref[...] * xc * lax.rsqrt(var + _EPS)
                     + bn0b_ref[...])
    degp = degp_ref[...]
    deg = degp[0:_N, 0:1] + degp[_NP:_NP + _N, 0:1] + 1.0
    dinv = lax.rsqrt(deg)
    m1 = jnp.dot(dinv * h0, c1w_ref[...], preferred_element_type=jnp.float32)
    h0_ref[...] = h0
    m1_ref[...] = m1
    dinv_ref[...] = dinv


_tc1 = pl.pallas_call(
    _tc1_body,
    out_shape=(
        jax.ShapeDtypeStruct((_N, _D), jnp.float32),
        jax.ShapeDtypeStruct((_N, _D), jnp.float32),
        jax.ShapeDtypeStruct((_N, 1), jnp.float32),
    ),
)


def _tc2_body(accp_ref, m1_ref, dinv_ref, h0_ref, c1b_ref, bn1g_ref,
              bn1b_ref, c2w_ref, m2_ref):
    a = accp_ref[...]
    agg = a[0:_N] + a[_NP:_NP + _N]
    dinv = dinv_ref[...]
    t = dinv * (agg + m1_ref[...]) + c1b_ref[...]
    mu = jnp.mean(t, axis=0, keepdims=True)
    xc = t - mu
    var = jnp.mean(xc * xc, axis=0, keepdims=True)
    h1 = jax.nn.relu(bn1g_ref[...] * xc * lax.rsqrt(var + _EPS)
                     + bn1b_ref[...]) + h0_ref[...]
    m2_ref[...] = jnp.dot(dinv * h1, c2w_ref[...],
                          preferred_element_type=jnp.float32)


_tc2 = pl.pallas_call(
    _tc2_body,
    out_shape=jax.ShapeDtypeStruct((_N, _D), jnp.float32),
)


def _tc3_body(accp_ref, m2_ref, dinv_ref, h0_ref, c2b_ref, bn2g_ref,
              bn2b_ref, out_ref):
    a = accp_ref[...]
    agg = a[0:_N] + a[_NP:_NP + _N]
    dinv = dinv_ref[...]
    t = dinv * (agg + m2_ref[...]) + c2b_ref[...]
    mu = jnp.mean(t, axis=0, keepdims=True)
    xc = t - mu
    var = jnp.mean(xc * xc, axis=0, keepdims=True)
    out_ref[...] = jax.nn.relu(bn2g_ref[...] * xc * lax.rsqrt(var + _EPS)
                               + bn2b_ref[...]) + h0_ref[...]


_tc3 = pl.pallas_call(
    _tc3_body,
    out_shape=jax.ShapeDtypeStruct((_N, _D), jnp.float32),
)


def kernel(x, edge_index, params):
    src = edge_index[0]
    dst = edge_index[1]
    pad = _EPAD - _E
    padi = jnp.arange(pad, dtype=jnp.int32)
    src2d = jnp.concatenate([src, padi % _TRASH]).reshape(_EROWS, _K)
    dst2d = jnp.concatenate(
        [dst, _N + (padi % _TRASH)]).reshape(_EROWS, _K)
    ones_m = jnp.ones((_K, _D), jnp.float32)
    zmat = jnp.zeros((_NP, _D), jnp.float32)

    p = params
    row = lambda v: v.reshape(1, _D)

    degp = _deg_kernel(dst2d, ones_m, zmat)
    h0, m1, dinv = _tc1(x, p['fc_w'], row(p['fc_b']), p['conv1_w'],
                        row(p['bn0_g']), row(p['bn0_b']), degp)
    acc1 = _scatter_kernel(m1, src2d, dst2d, zmat)
    m2 = _tc2(acc1, m1, dinv, h0, row(p['conv1_b']), row(p['bn1_g']),
              row(p['bn1_b']), p['conv2_w'])
    acc2 = _scatter_kernel(m2, src2d, dst2d, zmat)
    out = _tc3(acc2, m2, dinv, h0, row(p['conv2_b']), row(p['bn2_g']),
               row(p['bn2_b']))
    return out


# speedup vs baseline: 28.9143x; 1.1720x over previous
import functools

import jax
import jax.numpy as jnp
from jax import lax
from jax.experimental import pallas as pl
from jax.experimental.pallas import tpu as pltpu
from jax.experimental.pallas import tpu_sc as plsc

_N = 10000
_E = 320000
_D = 128
_EPS = 1e-5

_NC = 2
_NS = 16
_NW = _NC * _NS
_TRASH = 112
_NP = _N + _TRASH
_STRIPE = _NP // _NS
_K = 128
_RW = 80
_RH = 40
_EROWS = _NW * _RW
_EPAD = _EROWS * _K

_mesh = plsc.VectorSubcoreMesh(
    core_axis_name="c", subcore_axis_name="s",
    num_cores=_NC, num_subcores=_NS)


_HB = _K * _K


@functools.partial(
    pl.kernel,
    out_type=jax.ShapeDtypeStruct((_NW * _HB,), jnp.float32),
    mesh=_mesh,
    compiler_params=pltpu.CompilerParams(needs_layout_passes=False),
    scratch_types=[
        pltpu.VMEM((_RW * _K,), jnp.int32),
        pltpu.VMEM((_HB,), jnp.float32),
    ],
)
def _deg_kernel(dst_hbm, z_hbm, out_hbm, dst_v, hist):
    c = lax.axis_index("c")
    s = lax.axis_index("s")
    w = c * _NS + s
    pltpu.sync_copy(z_hbm, hist)
    pltpu.sync_copy(dst_hbm.at[pl.ds(w * _RW * _K, _RW * _K)], dst_v)

    def body(g, carry):
        for k in range(8):
            idx = dst_v[pl.ds((g * 8 + k) * 16, 16)]
            cnt, last = plsc.scan_count(idx)
            plsc.addupdate_scatter(hist, [idx], cnt.astype(jnp.float32),
                                   mask=last)
        return carry

    lax.fori_loop(0, _RW, body, 0)
    pltpu.sync_copy(hist, out_hbm.at[pl.ds(w * _HB, _HB)])


@functools.partial(
    pl.kernel,
    out_type=jax.ShapeDtypeStruct((_NC * _NP, _D), jnp.float32),
    mesh=_mesh,
    scratch_types=[
        pltpu.VMEM((_RH, _K), jnp.int32),
        pltpu.VMEM((_RH, _K), jnp.int32),
        pltpu.VMEM((_K, _D), jnp.float32),
        pltpu.VMEM((_K, _D), jnp.float32),
        pltpu.VMEM_SHARED((_NP, _D), jnp.float32),
        pltpu.SemaphoreType.DMA,
        pltpu.SemaphoreType.DMA,
        pltpu.SemaphoreType.DMA,
        pltpu.SemaphoreType.DMA,
    ],
)
def _scatter_kernel(m_hbm, src_hbm, dst_hbm, z_hbm, out_hbm,
                    src_v, dst_v, rows0, rows1, acc,
                    gsem0, gsem1, ssem0, ssem1):
    c = lax.axis_index("c")
    s = lax.axis_index("s")
    pltpu.sync_copy(z_hbm.at[pl.ds(s * _STRIPE, _STRIPE)],
                    acc.at[pl.ds(s * _STRIPE, _STRIPE)])
    w = c * _NS + s
    plsc.subcore_barrier()

    for h in range(_RW // _RH):
        base = w * _RW + h * _RH
        pltpu.sync_copy(src_hbm.at[pl.ds(base, _RH)], src_v)
        pltpu.sync_copy(dst_hbm.at[pl.ds(base, _RH)], dst_v)
        pltpu.async_copy(m_hbm.at[src_v.at[0]], rows0, gsem0)

        def pair(i, carry):
            g = 2 * i
            pltpu.make_async_copy(m_hbm.at[src_v.at[g]], rows0, gsem0).wait()
            pltpu.async_copy(rows0, acc.at[dst_v.at[g]], ssem0, add=True)

            @pl.when(i > 0)
            def _():
                pltpu.make_async_copy(rows1, acc.at[dst_v.at[g - 1]],
                                      ssem1).wait()

            pltpu.async_copy(m_hbm.at[src_v.at[g + 1]], rows1, gsem1)
            pltpu.make_async_copy(m_hbm.at[src_v.at[g + 1]], rows1,
                                  gsem1).wait()
            pltpu.async_copy(rows1, acc.at[dst_v.at[g + 1]], ssem1, add=True)
            pltpu.make_async_copy(rows0, acc.at[dst_v.at[g]], ssem0).wait()

            @pl.when(i < _RH // 2 - 1)
            def _():
                pltpu.async_copy(m_hbm.at[src_v.at[g + 2]], rows0, gsem0)

            return carry

        lax.fori_loop(0, _RH // 2, pair, 0)
        pltpu.make_async_copy(rows1, acc.at[dst_v.at[_RH - 1]], ssem1).wait()
    plsc.subcore_barrier()
    pltpu.sync_copy(acc.at[pl.ds(s * _STRIPE, _STRIPE)],
                    out_hbm.at[pl.ds(c * _NP + s * _STRIPE, _STRIPE)])


def _tc1a_body(x_ref, fcw_ref, fcb_ref, bn0g_ref, bn0b_ref, h0_ref):
    x = x_ref[...]
    h = jnp.dot(x, fcw_ref[...], preferred_element_type=jnp.float32)
    h = h + fcb_ref[...]
    mu = jnp.mean(h, axis=0, keepdims=True)
    xc = h - mu
    var = jnp.mean(xc * xc, axis=0, keepdims=True)
    h0_ref[...] = jax.nn.relu(bn0g_ref[...] * xc * lax.rsqrt(var + _EPS)
                              + bn0b_ref[...])


_tc1a = pl.pallas_call(
    _tc1a_body,
    out_shape=jax.ShapeDtypeStruct((_N, _D), jnp.float32),
)


def _tc1b_body(h0_ref, c1w_ref, degp_ref, m1_ref, dinv_ref):
    degp = degp_ref[...]
    dsum = jnp.sum(degp.reshape(_NW, _HB), axis=0)
    deg = dsum.reshape(_HB, 1)[0:_N] + 1.0
    dinv = lax.rsqrt(deg)
    m1 = jnp.dot(dinv * h0_ref[...], c1w_ref[...],
                 preferred_element_type=jnp.float32)
    m1_ref[...] = m1
    dinv_ref[...] = dinv


_tc1b = pl.pallas_call(
    _tc1b_body,
    out_shape=(
        jax.ShapeDtypeStruct((_N, _D), jnp.float32),
        jax.ShapeDtypeStruct((_N, 1), jnp.float32),
    ),
)


def _tc2_body(accp_ref, m1_ref, dinv_ref, h0_ref, c1b_ref, bn1g_ref,
              bn1b_ref, c2w_ref, m2_ref):
    a = accp_ref[...]
    agg = a[0:_N] + a[_NP:_NP + _N]
    dinv = dinv_ref[...]
    t = dinv * (agg + m1_ref[...]) + c1b_ref[...]
    mu = jnp.mean(t, axis=0, keepdims=True)
    xc = t - mu
    var = jnp.mean(xc * xc, axis=0, keepdims=True)
    h1 = jax.nn.relu(bn1g_ref[...] * xc * lax.rsqrt(var + _EPS)
                     + bn1b_ref[...]) + h0_ref[...]
    m2_ref[...] = jnp.dot(dinv * h1, c2w_ref[...],
                          preferred_element_type=jnp.float32)


_tc2 = pl.pallas_call(
    _tc2_body,
    out_shape=jax.ShapeDtypeStruct((_N, _D), jnp.float32),
)


def _tc3_body(accp_ref, m2_ref, dinv_ref, h0_ref, c2b_ref, bn2g_ref,
              bn2b_ref, out_ref):
    a = accp_ref[...]
    agg = a[0:_N] + a[_NP:_NP + _N]
    dinv = dinv_ref[...]
    t = dinv * (agg + m2_ref[...]) + c2b_ref[...]
    mu = jnp.mean(t, axis=0, keepdims=True)
    xc = t - mu
    var = jnp.mean(xc * xc, axis=0, keepdims=True)
    out_ref[...] = jax.nn.relu(bn2g_ref[...] * xc * lax.rsqrt(var + _EPS)
                               + bn2b_ref[...]) + h0_ref[...]


_tc3 = pl.pallas_call(
    _tc3_body,
    out_shape=jax.ShapeDtypeStruct((_N, _D), jnp.float32),
)


def kernel(x, edge_index, params):
    src = edge_index[0]
    dst = edge_index[1]
    pad = _EPAD - _E
    padi = jnp.arange(pad, dtype=jnp.int32)
    src2d = jnp.concatenate([src, padi % _TRASH]).reshape(_EROWS, _K)
    dst2d = jnp.concatenate(
        [dst, _N + (padi % _TRASH)]).reshape(_EROWS, _K)
    zmat = jnp.zeros((_NP, _D), jnp.float32)
    zflat = jnp.zeros((_HB,), jnp.float32)

    p = params
    row = lambda v: v.reshape(1, _D)

    degp = _deg_kernel(dst2d.reshape(_EPAD), zflat)
    h0 = _tc1a(x, p['fc_w'], row(p['fc_b']), row(p['bn0_g']),
               row(p['bn0_b']))
    m1, dinv = _tc1b(h0, p['conv1_w'], degp)
    acc1 = _scatter_kernel(m1, src2d, dst2d, zmat)
    m2 = _tc2(acc1, m1, dinv, h0, row(p['conv1_b']), row(p['bn1_g']),
              row(p['bn1_b']), p['conv2_w'])
    acc2 = _scatter_kernel(m2, src2d, dst2d, zmat)
    out = _tc3(acc2, m2, dinv, h0, row(p['conv2_b']), row(p['bn2_g']),
               row(p['bn2_b']))
    return out
